# Initial kernel scaffold; baseline (speedup 1.0000x reference)
import jax
import jax.numpy as jnp
from jax.experimental import pallas as pl


def kernel(loss, x, t):
    raise NotImplementedError("write your pallas kernel here")


# speedup vs baseline: 1.0279x; 1.0279x over previous
import functools

import jax
import jax.numpy as jnp
from jax import lax
from jax.experimental import pallas as pl
from jax.experimental.pallas import tpu as pltpu
from jax.experimental.pallas import tpu_sc as plsc

N = 1_000_000
NPAD = 1_048_576
NW = 32
C = NPAD // NW
S = 2048
G = 128
TRASH = NPAD
X_LO, X_HI = -1.0, 1.0
T_LO, T_HI = 0.0, 1.0

_mesh = plsc.VectorSubcoreMesh(core_axis_name="c", subcore_axis_name="s")


def _wid():
    return lax.axis_index("s") * 2 + lax.axis_index("c")


@functools.partial(
    pl.kernel,
    out_type=(
        jax.ShapeDtypeStruct((NPAD,), jnp.int32),
        jax.ShapeDtypeStruct((NW, 16), jnp.int32),
    ),
    mesh=_mesh,
    compiler_params=pltpu.CompilerParams(needs_layout_passes=False),
    scratch_types=[
        pltpu.VMEM((S,), jnp.float32),
        pltpu.VMEM((16,), jnp.float32),
        pltpu.VMEM((C + 16,), jnp.int32),
        pltpu.VMEM((16,), jnp.int32),
    ],
)
def _k2_compact(loss_hbm, mean_hbm, kidx_hbm, counts_hbm,
                loss_v, mean_v, kidx_v, cnt_v):
    w = _wid()
    base = w * C
    pltpu.sync_copy(mean_hbm, mean_v)
    meanv = mean_v[...]
    lane = lax.iota(jnp.int32, 16)

    def outer(f, wp):
        pltpu.sync_copy(loss_hbm.at[pl.ds(base + f * S, S)], loss_v)

        def inner(p, wpv):
            lv = loss_v[pl.ds(p * 16, 16)]
            m = lv > meanv
            gi = (base + f * S + p * 16) + lane
            pos = plsc.cumsum(m.astype(jnp.int32))
            plsc.store_scatter(kidx_v, [wpv + pos - 1], gi, mask=m)
            return wpv + plsc.all_reduce_population_count(m)

        return lax.fori_loop(0, S // 16, inner, wp)

    wp = lax.fori_loop(0, C // S, outer, jnp.zeros((16,), jnp.int32))
    pltpu.sync_copy(kidx_v.at[pl.ds(0, C)], kidx_hbm.at[pl.ds(base, C)])
    cnt_v[...] = wp
    pltpu.sync_copy(cnt_v, counts_hbm.at[w])


@functools.partial(
    pl.kernel,
    out_type=(
        jax.ShapeDtypeStruct((NPAD + 16,), jnp.float32),
        jax.ShapeDtypeStruct((NPAD + 16,), jnp.float32),
    ),
    mesh=_mesh,
    compiler_params=pltpu.CompilerParams(needs_layout_passes=False),
    scratch_types=[
        pltpu.VMEM((NW, 16), jnp.int32),
        pltpu.VMEM((G,), jnp.int32),
        pltpu.VMEM((G,), jnp.int32),
        pltpu.VMEM((G,), jnp.int32),
        pltpu.VMEM((G,), jnp.float32),
        pltpu.VMEM((G,), jnp.float32),
        pltpu.SemaphoreType.DMA,
    ],
)
def _k3_assemble(x_hbm, t_hbm, xn_hbm, tn_hbm, kidx_hbm, counts_hbm,
                 xo_hbm, to_hbm,
                 counts_v, kbuf, sidx, oidx, xbuf, tbuf, sem):
    w = _wid()
    base = w * C
    lane = lax.iota(jnp.int32, 16)
    zeros = jnp.zeros((16,), jnp.int32)

    pltpu.sync_copy(counts_hbm, counts_v)
    c0 = plsc.load_gather(counts_v, [lane, zeros])
    c1 = plsc.load_gather(counts_v, [lane + 16, zeros])
    s0 = plsc.cumsum(c0)
    s1 = plsc.cumsum(c1)
    tot0 = jnp.max(s0)
    total = tot0 + jnp.max(s1)
    e0 = s0 - c0
    e1 = (s1 - c1) + tot0
    lsel = jnp.where(w < 16, w, w - 16)
    ew = jnp.where(w < 16, e0, e1)
    cwv = jnp.where(w < 16, c0, c1)
    p_w = jnp.sum(jnp.where(lane == lsel, ew, 0))
    c_w = jnp.sum(jnp.where(lane == lsel, cwv, 0))

    def keep_body(q, _):
        @pl.when(q * G < c_w)
        def _():
            pltpu.sync_copy(kidx_hbm.at[pl.ds(base + q * G, G)], kbuf)
            for p in range(G // 16):
                kv = kbuf[pl.ds(p * 16, 16)]
                pos = (q * G + p * 16) + lane
                valid = pos < c_w
                sidx[pl.ds(p * 16, 16)] = jnp.where(valid, kv, 0)
                oidx[pl.ds(p * 16, 16)] = jnp.where(valid, p_w + pos,
                                                    TRASH + lane)
            gx = pltpu.async_copy(x_hbm.at[sidx], xbuf, sem)
            gt = pltpu.async_copy(t_hbm.at[sidx], tbuf, sem)
            gx.wait()
            gt.wait()
            wx = pltpu.async_copy(xbuf, xo_hbm.at[oidx], sem)
            wt = pltpu.async_copy(tbuf, to_hbm.at[oidx], sem)
            wx.wait()
            wt.wait()
        return 0

    lax.fori_loop(0, C // G, keep_body, 0)

    def tail_body(q, _):
        start = base + q * G

        @pl.when(start + G > total)
        def _():
            for p in range(G // 16):
                j = (start + p * 16) + lane
                tv = j >= total
                sidx[pl.ds(p * 16, 16)] = jnp.clip(j - total, 0, N - 1)
                oidx[pl.ds(p * 16, 16)] = jnp.where(tv, j, TRASH + lane)
            gx = pltpu.async_copy(xn_hbm.at[sidx], xbuf, sem)
            gt = pltpu.async_copy(tn_hbm.at[sidx], tbuf, sem)
            gx.wait()
            gt.wait()
            wx = pltpu.async_copy(xbuf, xo_hbm.at[oidx], sem)
            wt = pltpu.async_copy(tbuf, to_hbm.at[oidx], sem)
            wx.wait()
            wt.wait()
        return 0

    lax.fori_loop(0, C // G, tail_body, 0)


def kernel(loss, x, t):
    mean = jnp.mean(loss)
    mean_arr = jnp.full((16,), mean, jnp.float32)
    lf = jnp.pad(loss.reshape(-1), (0, NPAD - N), constant_values=-1.0)
    kn = jax.random.split(jax.random.key(1), 2)
    xn = jax.random.uniform(kn[0], (N, 1), minval=X_LO, maxval=X_HI,
                            dtype=jnp.float32).reshape(-1)
    tn = jax.random.uniform(kn[1], (N, 1), minval=T_LO, maxval=T_HI,
                            dtype=jnp.float32).reshape(-1)
    kidx, counts = _k2_compact(lf, mean_arr)
    xo, to = _k3_assemble(x.reshape(-1), t.reshape(-1), xn, tn, kidx, counts)
    return (xo[:N, None], to[:N, None])
